# Initial kernel scaffold; baseline (speedup 1.0000x reference)
import jax
import jax.numpy as jnp
from jax.experimental import pallas as pl


def kernel(onehots, tables):
    raise NotImplementedError("write your pallas kernel here")


# speedup vs baseline: 1.1468x; 1.1468x over previous
import functools

import jax
import jax.numpy as jnp
from jax import lax
from jax.experimental import pallas as pl
from jax.experimental.pallas import tpu as pltpu
from jax.experimental.pallas import tpu_sc as plsc

_F = 26
_V = 100000
_D = 16
_B = 16384
_ROWS = _B * _F
_NC, _NS = 2, 16
_NW = _NC * _NS
_PW = _ROWS // _NW
_C = 128
_J = _PW // _C
_G = 8
_NG = _J // _G

_mesh = plsc.VectorSubcoreMesh(core_axis_name="c", subcore_axis_name="s")


@functools.partial(
    pl.kernel,
    mesh=_mesh,
    out_type=jax.ShapeDtypeStruct((_ROWS, _D), jnp.float32),
    scratch_types=[
        pltpu.VMEM((_J, _C), jnp.int32),
        pltpu.VMEM((_G * _C, _D), jnp.float32),
        pltpu.SemaphoreType.DMA,
    ],
    compiler_params=pltpu.CompilerParams(use_tc_tiling_on_sc=False),
)
def _gather_rows(idx_hbm, tab_hbm, out_hbm, idx_v, rows_v, sem):
    wid = lax.axis_index("s") * _NC + lax.axis_index("c")
    pltpu.sync_copy(idx_hbm.at[pl.ds(wid * _J, _J)], idx_v)

    def group(g, carry):
        descs = [
            pltpu.async_copy(
                tab_hbm.at[idx_v.at[g * _G + j]],
                rows_v.at[pl.ds(j * _C, _C)],
                sem,
            )
            for j in range(_G)
        ]
        for d in descs:
            d.wait()
        pltpu.sync_copy(
            rows_v, out_hbm.at[pl.ds(wid * _PW + g * (_G * _C), _G * _C)]
        )
        return carry

    lax.fori_loop(0, _NG, group, 0)


def kernel(onehots, tables):
    flat_tab = tables.reshape(_F * _V, _D)
    offs = (jnp.arange(_F, dtype=jnp.int32) * _V)[None, :]
    idx = (onehots.astype(jnp.int32) + offs).reshape(_ROWS // _C, _C)
    out = _gather_rows(idx, flat_tab)
    return out.reshape(_B, _F * _D)
